# baseline (speedup 1.0000x reference)
import functools

import jax
import jax.numpy as jnp
from jax import lax
from jax.experimental import pallas as pl
from jax.experimental.pallas import tpu as pltpu
from jax.experimental.pallas import tpu_sc as plsc

N = 10000
D = 128
NC = 2
NS = 16
NW = NC * NS
G = 64
NP = 10240
SP = NP // NS


def _zero_rows(rows_v):
    def zrow(i, carry):
        for q in range(8):
            rows_v[i, pl.ds(q * 16, 16)] = jnp.zeros((16,), jnp.float32)
        return carry
    lax.fori_loop(0, G, zrow, 0)


SB = 8


def _agg_body(hs_hbm, src_hbm, dst_hbm, attr_hbm, out_hbm,
              src_v, dst_v, attr_v, rows_v, out_v, acc_sh, gsem, ssem, isem):
    c = lax.axis_index("c")
    s = lax.axis_index("s")
    wid = c * NS + s
    K = src_hbm.shape[0] // NW
    NSB = K // SB

    _zero_rows(out_v.at[0])
    base_r = s * SP
    for t in range(SP // G):
        pltpu.sync_copy(out_v.at[0], acc_sh.at[pl.ds(base_r + t * G, G)])
    plsc.subcore_barrier()

    idx_bufs = (src_v, dst_v, attr_v)

    def _stage(sbi, slot):
        gb = wid * K + sbi * SB
        for hb, vb in zip((src_hbm, dst_hbm, attr_hbm), idx_bufs):
            pltpu.async_copy(hb.at[pl.ds(gb, SB)], vb.at[slot], isem)

    def _stage_wait(sbi, slot):
        gb = wid * K + sbi * SB
        for hb, vb in zip((src_hbm, dst_hbm, attr_hbm), idx_bufs):
            pltpu.make_async_copy(hb.at[pl.ds(gb, SB)], vb.at[slot],
                                  isem).wait()

    def _gather(slot, gl, b):
        pltpu.async_copy(hs_hbm.at[src_v.at[slot, gl]], rows_v.at[b], gsem)

    def _gather_wait(slot, gl, b):
        pltpu.make_async_copy(hs_hbm.at[src_v.at[slot, gl]], rows_v.at[b],
                              gsem).wait()

    def _scatter(slot, gl, b):
        pltpu.async_copy(out_v.at[b], acc_sh.at[dst_v.at[slot, gl]], ssem,
                         add=True)

    def _scatter_wait(slot, gl, b):
        pltpu.make_async_copy(out_v.at[b], acc_sh.at[dst_v.at[slot, gl]],
                              ssem).wait()

    _stage(0, 0)
    _stage_wait(0, 0)
    _gather(0, 0, 0)
    _gather(0, 1, 1)

    def sb_pair(p, carry):
        for par in range(2):
            cur = par
            nxt = 1 - par
            sbi = p * 2 + par

            @pl.when(sbi + 1 < NSB)
            def _prefetch(_c=cur, _n=nxt):
                _stage(sbi + 1, _n)

            def inner(it, c2, _cur=cur, _nxt=nxt, _sbi=sbi):
                for b2 in range(2):
                    gl = it * 2 + b2
                    g = _sbi * SB + gl
                    gbuf = lax.rem(g, 3)
                    gnbuf = lax.rem(g + 2, 3)

                    @pl.when(jnp.logical_and(g >= 2, gl >= 2))
                    def _drain_same(_b=b2, _cur=_cur, _gl=gl):
                        _scatter_wait(_cur, _gl - 2, _b)

                    @pl.when(jnp.logical_and(g >= 2, gl < 2))
                    def _drain_prev(_b=b2, _nxt=_nxt, _gl=gl):
                        _scatter_wait(_nxt, _gl + SB - 2, _b)

                    @pl.when(it < SB // 2 - 1)
                    def _issue_same(_cur=_cur, _gl=gl, _b=gnbuf):
                        _gather(_cur, _gl + 2, _b)

                    @pl.when(jnp.logical_and(it == SB // 2 - 1,
                                             _sbi + 1 < NSB))
                    def _issue_next(_nxt=_nxt, _b=gnbuf, _b2=b2, _sbi=_sbi):
                        if _b2 == 0:
                            _stage_wait(_sbi + 1, _nxt)
                        _gather(_nxt, _b2, _b)

                    _gather_wait(_cur, gl, gbuf)

                    def scale16(q16, c3, _b=b2, _gl=gl, _cur=_cur,
                                _gb=gbuf):
                        wv = attr_v[_cur, _gl, pl.ds(q16 * 16, 16)]
                        for l in range(16):
                            w = wv[l]
                            e = q16 * 16 + l
                            for q in range(8):
                                sl = pl.ds(q * 16, 16)
                                out_v[_b, e, sl] = rows_v[_gb, e, sl] * w
                        return c3
                    lax.fori_loop(0, G // 16, scale16, 0)

                    _scatter(_cur, gl, b2)
                return c2
            lax.fori_loop(0, SB // 2, inner, 0)
        return carry
    lax.fori_loop(0, NSB // 2, sb_pair, 0)

    last_slot = (NSB - 1) % 2
    for gl in (SB - 2, SB - 1):
        _scatter_wait(last_slot, gl, gl % 2)

    plsc.subcore_barrier()
    n_out = N - (NS - 1) * SP

    @pl.when(s < NS - 1)
    def _full_out():
        pltpu.sync_copy(acc_sh.at[pl.ds(base_r, SP)],
                        out_hbm.at[c, pl.ds(base_r, SP)])

    @pl.when(s == NS - 1)
    def _tail_out():
        tail = (NS - 1) * SP
        pltpu.sync_copy(acc_sh.at[pl.ds(tail, n_out)],
                        out_hbm.at[c, pl.ds(tail, n_out)])


def _aggregate(hs, src2, dst2, attr2):
    K = src2.shape[0] // NW
    kern = pl.kernel(
        _agg_body,
        out_type=jax.ShapeDtypeStruct((NC, N, D), jnp.float32),
        mesh=plsc.VectorSubcoreMesh(core_axis_name="c", subcore_axis_name="s"),
        scratch_types=[
            pltpu.VMEM((2, SB, G), jnp.int32),
            pltpu.VMEM((2, SB, G), jnp.int32),
            pltpu.VMEM((2, SB, G), jnp.float32),
            pltpu.VMEM((3, G, D), jnp.float32),
            pltpu.VMEM((2, G, D), jnp.float32),
            pltpu.VMEM_SHARED((NP, D), jnp.float32),
            pltpu.SemaphoreType.DMA,
            pltpu.SemaphoreType.DMA,
            pltpu.SemaphoreType.DMA,
        ],
    )
    return kern(hs, src2, dst2, attr2)


def _deg_body(dst_hbm, attr_hbm, out_hbm, dst_v, attr_v, zb_v, acc_sh):
    c = lax.axis_index("c")
    s = lax.axis_index("s")
    wid = c * NS + s
    K = dst_hbm.shape[0] // NW

    @pl.when(s == 0)
    def _init():
        def z(i, carry):
            zb_v[pl.ds(i * 16, 16)] = jnp.zeros((16,), jnp.float32)
            return carry
        lax.fori_loop(0, N // 16, z, 0)
        pltpu.sync_copy(zb_v, acc_sh)
    plsc.subcore_barrier()

    pltpu.sync_copy(dst_hbm.at[pl.ds(wid * K, K)], dst_v)
    pltpu.sync_copy(attr_hbm.at[pl.ds(wid * K, K)], attr_v)

    def group(j, carry):
        pltpu.sync_copy(attr_v.at[j], acc_sh.at[dst_v.at[j]], add=True)
        return carry
    lax.fori_loop(0, K, group, 0)

    plsc.subcore_barrier()

    @pl.when(s == 0)
    def _out():
        pltpu.sync_copy(acc_sh, out_hbm.at[c])


def _degree(dst2, attr2):
    K = dst2.shape[0] // NW
    kern = pl.kernel(
        _deg_body,
        out_type=jax.ShapeDtypeStruct((NC, N), jnp.float32),
        mesh=plsc.VectorSubcoreMesh(core_axis_name="c", subcore_axis_name="s"),
        scratch_types=[
            pltpu.VMEM((K, G), jnp.int32),
            pltpu.VMEM((K, G), jnp.float32),
            pltpu.VMEM((N,), jnp.float32),
            pltpu.VMEM_SHARED((N,), jnp.float32),
        ],
    )
    return kern(dst2, attr2)


_R = 1000


def _mm1_body(x_ref, w_ref, dv_ref, h_ref, hs_ref):
    h = jnp.dot(x_ref[...], w_ref[...], preferred_element_type=jnp.float32)
    h_ref[...] = h
    hs_ref[...] = h * dv_ref[...]


def _mm1(x, W, dinvb):
    return pl.pallas_call(
        _mm1_body,
        grid=(N // _R,),
        in_specs=[pl.BlockSpec((_R, D), lambda i: (i, 0)),
                  pl.BlockSpec((D, D), lambda i: (0, 0)),
                  pl.BlockSpec((_R, D), lambda i: (i, 0))],
        out_specs=[pl.BlockSpec((_R, D), lambda i: (i, 0)),
                   pl.BlockSpec((_R, D), lambda i: (i, 0))],
        out_shape=[jax.ShapeDtypeStruct((N, D), jnp.float32),
                   jax.ShapeDtypeStruct((N, D), jnp.float32)],
    )(x, W, dinvb)


def _mid_body(s_ref, h_ref, dv_ref, b_ref, w_ref, ho_ref, hso_ref):
    dv = dv_ref[...]
    xin = jnp.maximum(
        dv * (s_ref[0] + s_ref[1]) + 2.0 * dv * dv * h_ref[...] + b_ref[...],
        0.0)
    h = jnp.dot(xin, w_ref[...], preferred_element_type=jnp.float32)
    ho_ref[...] = h
    hso_ref[...] = h * dv


def _mid(s_p, h_prev, dinvb, b, W):
    return pl.pallas_call(
        _mid_body,
        grid=(N // _R,),
        in_specs=[pl.BlockSpec((NC, _R, D), lambda i: (0, i, 0)),
                  pl.BlockSpec((_R, D), lambda i: (i, 0)),
                  pl.BlockSpec((_R, D), lambda i: (i, 0)),
                  pl.BlockSpec((1, D), lambda i: (0, 0)),
                  pl.BlockSpec((D, D), lambda i: (0, 0))],
        out_specs=[pl.BlockSpec((_R, D), lambda i: (i, 0)),
                   pl.BlockSpec((_R, D), lambda i: (i, 0))],
        out_shape=[jax.ShapeDtypeStruct((N, D), jnp.float32),
                   jax.ShapeDtypeStruct((N, D), jnp.float32)],
    )(s_p, h_prev, dinvb, b, W)


def _final_body(s_ref, h_ref, dv_ref, b_ref, o_ref):
    dv = dv_ref[...]
    o_ref[...] = (dv * (s_ref[0] + s_ref[1])
                  + 2.0 * dv * dv * h_ref[...] + b_ref[...])


def _final(s_p, h_prev, dinvb, b):
    return pl.pallas_call(
        _final_body,
        grid=(N // _R,),
        in_specs=[pl.BlockSpec((NC, _R, D), lambda i: (0, i, 0)),
                  pl.BlockSpec((_R, D), lambda i: (i, 0)),
                  pl.BlockSpec((_R, D), lambda i: (i, 0)),
                  pl.BlockSpec((1, D), lambda i: (0, 0))],
        out_specs=pl.BlockSpec((_R, D), lambda i: (i, 0)),
        out_shape=jax.ShapeDtypeStruct((N, D), jnp.float32),
    )(s_p, h_prev, dinvb, b)


def kernel(x, edge_index, edge_attr, W1, b1, W2, b2, W3, b3):
    src = edge_index[0].astype(jnp.int32)
    dst = edge_index[1].astype(jnp.int32)
    E = src.shape[0]
    ep = NW * G * 8
    EP = ((E + ep - 1) // ep) * ep
    padn = EP - E
    if padn:
        pad_idx = jnp.arange(padn, dtype=jnp.int32) % N
        src = jnp.concatenate([src, pad_idx])
        dst = jnp.concatenate([dst, pad_idx])
        ea = jnp.concatenate([edge_attr.astype(jnp.float32),
                              jnp.zeros((padn,), jnp.float32)])
    else:
        ea = edge_attr.astype(jnp.float32)
    src2 = src.reshape(-1, G)
    dst2 = dst.reshape(-1, G)
    attr2 = ea.reshape(-1, G)

    degp = _degree(dst2, attr2)
    deg = degp[0] + degp[1] + 2.0
    dinv = jnp.where(deg > 0, jax.lax.rsqrt(deg), 0.0)
    dinvb = jnp.broadcast_to(dinv[:, None], (N, D))

    b1r = b1.reshape(1, D)
    b2r = b2.reshape(1, D)
    b3r = b3.reshape(1, D)

    h1, hs1 = _mm1(x, W1, dinvb)
    s1 = _aggregate(hs1, src2, dst2, attr2)
    h2, hs2 = _mid(s1, h1, dinvb, b1r, W2)
    s2 = _aggregate(hs2, src2, dst2, attr2)
    h3, hs3 = _mid(s2, h2, dinvb, b2r, W3)
    s3 = _aggregate(hs3, src2, dst2, attr2)
    return _final(s3, h3, dinvb, b3r)


# speedup vs baseline: 3.7925x; 3.7925x over previous
import functools

import jax
import jax.numpy as jnp
from jax import lax
from jax.experimental import pallas as pl
from jax.experimental.pallas import tpu as pltpu
from jax.experimental.pallas import tpu_sc as plsc

N = 10000
D = 128
NC = 2
NS = 16
NW = NC * NS
G = 64
NP = 10240
SP = NP // NS


def _zero_rows(rows_v):
    def zrow(i, carry):
        for q in range(8):
            rows_v[i, pl.ds(q * 16, 16)] = jnp.zeros((16,), jnp.float32)
        return carry
    lax.fori_loop(0, G, zrow, 0)


SB = 8


def _agg_body(hs_hbm, src_hbm, dst_hbm, attr_hbm, out_hbm,
              src_v, dst_v, attr_v, rows_v, acc_sh, gsem, ssem, isem):
    c = lax.axis_index("c")
    s = lax.axis_index("s")
    wid = c * NS + s
    K = src_hbm.shape[0] // NW
    NSB = K // SB

    _zero_rows(rows_v.at[0])
    base_r = s * SP
    for t in range(SP // G):
        pltpu.sync_copy(rows_v.at[0], acc_sh.at[pl.ds(base_r + t * G, G)])
    plsc.subcore_barrier()

    idx_bufs = (src_v, dst_v, attr_v)

    def _stage(sbi, slot):
        gb = wid * K + sbi * SB
        for hb, vb in zip((src_hbm, dst_hbm, attr_hbm), idx_bufs):
            pltpu.async_copy(hb.at[pl.ds(gb, SB)], vb.at[slot], isem)

    def _stage_wait(sbi, slot):
        gb = wid * K + sbi * SB
        for hb, vb in zip((src_hbm, dst_hbm, attr_hbm), idx_bufs):
            pltpu.make_async_copy(hb.at[pl.ds(gb, SB)], vb.at[slot],
                                  isem).wait()

    def _gather(slot, gl, b):
        pltpu.async_copy(hs_hbm.at[src_v.at[slot, gl]], rows_v.at[b], gsem)

    def _gather_wait(slot, gl, b):
        pltpu.make_async_copy(hs_hbm.at[src_v.at[slot, gl]], rows_v.at[b],
                              gsem).wait()

    def _scatter(slot, gl, b):
        pltpu.async_copy(rows_v.at[b], acc_sh.at[dst_v.at[slot, gl]], ssem,
                         add=True)

    def _scatter_wait(slot, gl, b):
        pltpu.make_async_copy(rows_v.at[b], acc_sh.at[dst_v.at[slot, gl]],
                              ssem).wait()

    _stage(0, 0)
    _stage_wait(0, 0)
    _gather(0, 0, 0)
    _gather(0, 1, 1)

    def sb_pair(p, carry):
        for par in range(2):
            cur = par
            nxt = 1 - par
            sbi = p * 2 + par

            @pl.when(sbi + 1 < NSB)
            def _prefetch(_c=cur, _n=nxt):
                _stage(sbi + 1, _n)

            def inner(it, c2, _cur=cur, _nxt=nxt, _sbi=sbi):
                for u in range(4):
                    gl = it * 4 + u
                    g = _sbi * SB + gl
                    bd = (u + 2) % 4

                    if u < 2:
                        @pl.when(jnp.logical_and(g >= 2, gl >= 2))
                        def _drain_same(_b=bd, _cur=_cur, _gl=gl):
                            _scatter_wait(_cur, _gl - 2, _b)

                        @pl.when(jnp.logical_and(g >= 2, gl < 2))
                        def _drain_prev(_b=bd, _nxt=_nxt, _gl=gl):
                            _scatter_wait(_nxt, _gl + SB - 2, _b)
                    else:
                        @pl.when(g >= 2)
                        def _drain(_b=bd, _cur=_cur, _gl=gl):
                            _scatter_wait(_cur, _gl - 2, _b)

                    if u < 2:
                        @pl.when(g + 2 < K)
                        def _issue_same(_cur=_cur, _gl=gl, _b=bd):
                            _gather(_cur, _gl + 2, _b)
                    else:
                        @pl.when(it < SB // 4 - 1)
                        def _issue_same2(_cur=_cur, _gl=gl, _b=bd):
                            _gather(_cur, _gl + 2, _b)

                        @pl.when(jnp.logical_and(it == SB // 4 - 1,
                                                 _sbi + 1 < NSB))
                        def _issue_next(_nxt=_nxt, _b=bd, _u=u, _sbi=_sbi):
                            if _u == 2:
                                _stage_wait(_sbi + 1, _nxt)
                            _gather(_nxt, _u - 2, _b)

                    _gather_wait(_cur, gl, u)

                    def scale16(q16, c3, _b=u, _gl=gl, _cur=_cur):
                        wv = attr_v[_cur, _gl, pl.ds(q16 * 16, 16)]
                        for l in range(16):
                            w = wv[l]
                            e = q16 * 16 + l
                            for q in range(8):
                                sl = pl.ds(q * 16, 16)
                                rows_v[_b, e, sl] = rows_v[_b, e, sl] * w
                        return c3
                    lax.fori_loop(0, G // 16, scale16, 0)

                    _scatter(_cur, gl, u)
                return c2
            lax.fori_loop(0, SB // 4, inner, 0)
        return carry
    lax.fori_loop(0, NSB // 2, sb_pair, 0)

    last_slot = (NSB - 1) % 2
    for gl in (SB - 2, SB - 1):
        _scatter_wait(last_slot, gl, gl % 4)

    plsc.subcore_barrier()
    n_out = N - (NS - 1) * SP

    @pl.when(s < NS - 1)
    def _full_out():
        pltpu.sync_copy(acc_sh.at[pl.ds(base_r, SP)],
                        out_hbm.at[c, pl.ds(base_r, SP)])

    @pl.when(s == NS - 1)
    def _tail_out():
        tail = (NS - 1) * SP
        pltpu.sync_copy(acc_sh.at[pl.ds(tail, n_out)],
                        out_hbm.at[c, pl.ds(tail, n_out)])


def _aggregate(hs, src2, dst2, attr2):
    K = src2.shape[0] // NW
    kern = pl.kernel(
        _agg_body,
        out_type=jax.ShapeDtypeStruct((NC, N, D), jnp.float32),
        mesh=plsc.VectorSubcoreMesh(core_axis_name="c", subcore_axis_name="s"),
        scratch_types=[
            pltpu.VMEM((2, SB, G), jnp.int32),
            pltpu.VMEM((2, SB, G), jnp.int32),
            pltpu.VMEM((2, SB, G), jnp.float32),
            pltpu.VMEM((4, G, D), jnp.float32),
            pltpu.VMEM_SHARED((NP, D), jnp.float32),
            pltpu.SemaphoreType.DMA,
            pltpu.SemaphoreType.DMA,
            pltpu.SemaphoreType.DMA,
        ],
    )
    return kern(hs, src2, dst2, attr2)


def _deg_body(dst_hbm, attr_hbm, out_hbm, dst_v, attr_v, zb_v, acc_sh):
    c = lax.axis_index("c")
    s = lax.axis_index("s")
    wid = c * NS + s
    K = dst_hbm.shape[0] // NW

    @pl.when(s == 0)
    def _init():
        def z(i, carry):
            zb_v[pl.ds(i * 16, 16)] = jnp.zeros((16,), jnp.float32)
            return carry
        lax.fori_loop(0, N // 16, z, 0)
        pltpu.sync_copy(zb_v, acc_sh)
    plsc.subcore_barrier()

    pltpu.sync_copy(dst_hbm.at[pl.ds(wid * K, K)], dst_v)
    pltpu.sync_copy(attr_hbm.at[pl.ds(wid * K, K)], attr_v)

    def group(j, carry):
        pltpu.sync_copy(attr_v.at[j], acc_sh.at[dst_v.at[j]], add=True)
        return carry
    lax.fori_loop(0, K, group, 0)

    plsc.subcore_barrier()

    @pl.when(s == 0)
    def _out():
        pltpu.sync_copy(acc_sh, out_hbm.at[c])


def _degree(dst2, attr2):
    K = dst2.shape[0] // NW
    kern = pl.kernel(
        _deg_body,
        out_type=jax.ShapeDtypeStruct((NC, N), jnp.float32),
        mesh=plsc.VectorSubcoreMesh(core_axis_name="c", subcore_axis_name="s"),
        scratch_types=[
            pltpu.VMEM((K, G), jnp.int32),
            pltpu.VMEM((K, G), jnp.float32),
            pltpu.VMEM((N,), jnp.float32),
            pltpu.VMEM_SHARED((N,), jnp.float32),
        ],
    )
    return kern(dst2, attr2)


_R = 1000


def _mm1_body(x_ref, w_ref, dv_ref, h_ref, hs_ref):
    h = jnp.dot(x_ref[...], w_ref[...], preferred_element_type=jnp.float32)
    h_ref[...] = h
    hs_ref[...] = h * dv_ref[...]


def _mm1(x, W, dinvb):
    return pl.pallas_call(
        _mm1_body,
        grid=(N // _R,),
        in_specs=[pl.BlockSpec((_R, D), lambda i: (i, 0)),
                  pl.BlockSpec((D, D), lambda i: (0, 0)),
                  pl.BlockSpec((_R, D), lambda i: (i, 0))],
        out_specs=[pl.BlockSpec((_R, D), lambda i: (i, 0)),
                   pl.BlockSpec((_R, D), lambda i: (i, 0))],
        out_shape=[jax.ShapeDtypeStruct((N, D), jnp.float32),
                   jax.ShapeDtypeStruct((N, D), jnp.float32)],
    )(x, W, dinvb)


def _mid_body(s_ref, h_ref, dv_ref, b_ref, w_ref, ho_ref, hso_ref):
    dv = dv_ref[...]
    xin = jnp.maximum(
        dv * (s_ref[0] + s_ref[1]) + 2.0 * dv * dv * h_ref[...] + b_ref[...],
        0.0)
    h = jnp.dot(xin, w_ref[...], preferred_element_type=jnp.float32)
    ho_ref[...] = h
    hso_ref[...] = h * dv


def _mid(s_p, h_prev, dinvb, b, W):
    return pl.pallas_call(
        _mid_body,
        grid=(N // _R,),
        in_specs=[pl.BlockSpec((NC, _R, D), lambda i: (0, i, 0)),
                  pl.BlockSpec((_R, D), lambda i: (i, 0)),
                  pl.BlockSpec((_R, D), lambda i: (i, 0)),
                  pl.BlockSpec((1, D), lambda i: (0, 0)),
                  pl.BlockSpec((D, D), lambda i: (0, 0))],
        out_specs=[pl.BlockSpec((_R, D), lambda i: (i, 0)),
                   pl.BlockSpec((_R, D), lambda i: (i, 0))],
        out_shape=[jax.ShapeDtypeStruct((N, D), jnp.float32),
                   jax.ShapeDtypeStruct((N, D), jnp.float32)],
    )(s_p, h_prev, dinvb, b, W)


def _final_body(s_ref, h_ref, dv_ref, b_ref, o_ref):
    dv = dv_ref[...]
    o_ref[...] = (dv * (s_ref[0] + s_ref[1])
                  + 2.0 * dv * dv * h_ref[...] + b_ref[...])


def _final(s_p, h_prev, dinvb, b):
    return pl.pallas_call(
        _final_body,
        grid=(N // _R,),
        in_specs=[pl.BlockSpec((NC, _R, D), lambda i: (0, i, 0)),
                  pl.BlockSpec((_R, D), lambda i: (i, 0)),
                  pl.BlockSpec((_R, D), lambda i: (i, 0)),
                  pl.BlockSpec((1, D), lambda i: (0, 0))],
        out_specs=pl.BlockSpec((_R, D), lambda i: (i, 0)),
        out_shape=jax.ShapeDtypeStruct((N, D), jnp.float32),
    )(s_p, h_prev, dinvb, b)


def kernel(x, edge_index, edge_attr, W1, b1, W2, b2, W3, b3):
    src = edge_index[0].astype(jnp.int32)
    dst = edge_index[1].astype(jnp.int32)
    E = src.shape[0]
    ep = NW * G * 8
    EP = ((E + ep - 1) // ep) * ep
    padn = EP - E
    if padn:
        pad_idx = jnp.arange(padn, dtype=jnp.int32) % N
        src = jnp.concatenate([src, pad_idx])
        dst = jnp.concatenate([dst, pad_idx])
        ea = jnp.concatenate([edge_attr.astype(jnp.float32),
                              jnp.zeros((padn,), jnp.float32)])
    else:
        ea = edge_attr.astype(jnp.float32)
    src2 = src.reshape(-1, G)
    dst2 = dst.reshape(-1, G)
    attr2 = ea.reshape(-1, G)

    degp = _degree(dst2, attr2)
    deg = degp[0] + degp[1] + 2.0
    dinv = jnp.where(deg > 0, jax.lax.rsqrt(deg), 0.0)
    dinvb = jnp.broadcast_to(dinv[:, None], (N, D))

    b1r = b1.reshape(1, D)
    b2r = b2.reshape(1, D)
    b3r = b3.reshape(1, D)

    h1, hs1 = _mm1(x, W1, dinvb)
    s1 = _aggregate(hs1, src2, dst2, attr2)
    h2, hs2 = _mid(s1, h1, dinvb, b1r, W2)
    s2 = _aggregate(hs2, src2, dst2, attr2)
    h3, hs3 = _mid(s2, h2, dinvb, b2r, W3)
    s3 = _aggregate(hs3, src2, dst2, attr2)
    return _final(s3, h3, dinvb, b3r)


# speedup vs baseline: 3.9168x; 1.0328x over previous
import functools

import jax
import jax.numpy as jnp
from jax import lax
from jax.experimental import pallas as pl
from jax.experimental.pallas import tpu as pltpu
from jax.experimental.pallas import tpu_sc as plsc

N = 10000
D = 128
NC = 2
NS = 16
NW = NC * NS
G = 64
NP = 10240
SP = NP // NS


def _zero_rows(rows_v):
    def zrow(i, carry):
        for q in range(8):
            rows_v[i, pl.ds(q * 16, 16)] = jnp.zeros((16,), jnp.float32)
        return carry
    lax.fori_loop(0, G, zrow, 0)


SB = 16


def _agg_body(hs_hbm, src_hbm, dst_hbm, attr_hbm, out_hbm,
              src_v, dst_v, attr_v, rows_v, acc_sh, gsem, ssem, isem):
    c = lax.axis_index("c")
    s = lax.axis_index("s")
    wid = c * NS + s
    K = src_hbm.shape[0] // NW
    NSB = K // SB

    _zero_rows(rows_v.at[0])
    base_r = s * SP
    for t in range(SP // G):
        pltpu.sync_copy(rows_v.at[0], acc_sh.at[pl.ds(base_r + t * G, G)])
    plsc.subcore_barrier()

    idx_bufs = (src_v, dst_v, attr_v)

    def _stage(sbi, slot):
        gb = wid * K + sbi * SB
        for hb, vb in zip((src_hbm, dst_hbm, attr_hbm), idx_bufs):
            pltpu.async_copy(hb.at[pl.ds(gb, SB)], vb.at[slot], isem)

    def _stage_wait(sbi, slot):
        gb = wid * K + sbi * SB
        for hb, vb in zip((src_hbm, dst_hbm, attr_hbm), idx_bufs):
            pltpu.make_async_copy(hb.at[pl.ds(gb, SB)], vb.at[slot],
                                  isem).wait()

    def _gather(slot, gl, b):
        pltpu.async_copy(hs_hbm.at[src_v.at[slot, gl]], rows_v.at[b], gsem)

    def _gather_wait(slot, gl, b):
        pltpu.make_async_copy(hs_hbm.at[src_v.at[slot, gl]], rows_v.at[b],
                              gsem).wait()

    def _scatter(slot, gl, b):
        pltpu.async_copy(rows_v.at[b], acc_sh.at[dst_v.at[slot, gl]], ssem,
                         add=True)

    def _scatter_wait(slot, gl, b):
        pltpu.make_async_copy(rows_v.at[b], acc_sh.at[dst_v.at[slot, gl]],
                              ssem).wait()

    _stage(0, 0)
    _stage_wait(0, 0)
    _gather(0, 0, 0)
    _gather(0, 1, 1)

    def sb_pair(p, carry):
        for par in range(2):
            cur = par
            nxt = 1 - par
            sbi = p * 2 + par

            @pl.when(sbi + 1 < NSB)
            def _prefetch(_c=cur, _n=nxt):
                _stage(sbi + 1, _n)

            def inner(it, c2, _cur=cur, _nxt=nxt, _sbi=sbi):
                for u in range(4):
                    gl = it * 4 + u
                    g = _sbi * SB + gl
                    bd = (u + 2) % 4

                    if u < 2:
                        @pl.when(jnp.logical_and(g >= 2, gl >= 2))
                        def _drain_same(_b=bd, _cur=_cur, _gl=gl):
                            _scatter_wait(_cur, _gl - 2, _b)

                        @pl.when(jnp.logical_and(g >= 2, gl < 2))
                        def _drain_prev(_b=bd, _nxt=_nxt, _gl=gl):
                            _scatter_wait(_nxt, _gl + SB - 2, _b)
                    else:
                        @pl.when(g >= 2)
                        def _drain(_b=bd, _cur=_cur, _gl=gl):
                            _scatter_wait(_cur, _gl - 2, _b)

                    if u < 2:
                        @pl.when(g + 2 < K)
                        def _issue_same(_cur=_cur, _gl=gl, _b=bd):
                            _gather(_cur, _gl + 2, _b)
                    else:
                        @pl.when(it < SB // 4 - 1)
                        def _issue_same2(_cur=_cur, _gl=gl, _b=bd):
                            _gather(_cur, _gl + 2, _b)

                        @pl.when(jnp.logical_and(it == SB // 4 - 1,
                                                 _sbi + 1 < NSB))
                        def _issue_next(_nxt=_nxt, _b=bd, _u=u, _sbi=_sbi):
                            if _u == 2:
                                _stage_wait(_sbi + 1, _nxt)
                            _gather(_nxt, _u - 2, _b)

                    _gather_wait(_cur, gl, u)

                    def scale16(q16, c3, _b=u, _gl=gl, _cur=_cur):
                        wv = attr_v[_cur, _gl, pl.ds(q16 * 16, 16)]
                        for l in range(16):
                            w = wv[l]
                            e = q16 * 16 + l
                            for q in range(8):
                                sl = pl.ds(q * 16, 16)
                                rows_v[_b, e, sl] = rows_v[_b, e, sl] * w
                        return c3
                    lax.fori_loop(0, G // 16, scale16, 0, unroll=2)

                    _scatter(_cur, gl, u)
                return c2
            lax.fori_loop(0, SB // 4, inner, 0)
        return carry
    lax.fori_loop(0, NSB // 2, sb_pair, 0)

    last_slot = (NSB - 1) % 2
    for gl in (SB - 2, SB - 1):
        _scatter_wait(last_slot, gl, gl % 4)

    plsc.subcore_barrier()
    n_out = N - (NS - 1) * SP

    @pl.when(s < NS - 1)
    def _full_out():
        pltpu.sync_copy(acc_sh.at[pl.ds(base_r, SP)],
                        out_hbm.at[c, pl.ds(base_r, SP)])

    @pl.when(s == NS - 1)
    def _tail_out():
        tail = (NS - 1) * SP
        pltpu.sync_copy(acc_sh.at[pl.ds(tail, n_out)],
                        out_hbm.at[c, pl.ds(tail, n_out)])


def _aggregate(hs, src2, dst2, attr2):
    K = src2.shape[0] // NW
    kern = pl.kernel(
        _agg_body,
        out_type=jax.ShapeDtypeStruct((NC, N, D), jnp.float32),
        mesh=plsc.VectorSubcoreMesh(core_axis_name="c", subcore_axis_name="s"),
        scratch_types=[
            pltpu.VMEM((2, SB, G), jnp.int32),
            pltpu.VMEM((2, SB, G), jnp.int32),
            pltpu.VMEM((2, SB, G), jnp.float32),
            pltpu.VMEM((4, G, D), jnp.float32),
            pltpu.VMEM_SHARED((NP, D), jnp.float32),
            pltpu.SemaphoreType.DMA,
            pltpu.SemaphoreType.DMA,
            pltpu.SemaphoreType.DMA,
        ],
    )
    return kern(hs, src2, dst2, attr2)


def _deg_body(dst_hbm, attr_hbm, out_hbm, dst_v, attr_v, zb_v, acc_sh):
    c = lax.axis_index("c")
    s = lax.axis_index("s")
    wid = c * NS + s
    K = dst_hbm.shape[0] // NW

    @pl.when(s == 0)
    def _init():
        def z(i, carry):
            zb_v[pl.ds(i * 16, 16)] = jnp.zeros((16,), jnp.float32)
            return carry
        lax.fori_loop(0, N // 16, z, 0)
        pltpu.sync_copy(zb_v, acc_sh)
    plsc.subcore_barrier()

    pltpu.sync_copy(dst_hbm.at[pl.ds(wid * K, K)], dst_v)
    pltpu.sync_copy(attr_hbm.at[pl.ds(wid * K, K)], attr_v)

    def group(j, carry):
        pltpu.sync_copy(attr_v.at[j], acc_sh.at[dst_v.at[j]], add=True)
        return carry
    lax.fori_loop(0, K, group, 0)

    plsc.subcore_barrier()

    @pl.when(s == 0)
    def _out():
        pltpu.sync_copy(acc_sh, out_hbm.at[c])


def _degree(dst2, attr2):
    K = dst2.shape[0] // NW
    GD = dst2.shape[1]
    kern = pl.kernel(
        _deg_body,
        out_type=jax.ShapeDtypeStruct((NC, N), jnp.float32),
        mesh=plsc.VectorSubcoreMesh(core_axis_name="c", subcore_axis_name="s"),
        scratch_types=[
            pltpu.VMEM((K, GD), jnp.int32),
            pltpu.VMEM((K, GD), jnp.float32),
            pltpu.VMEM((N,), jnp.float32),
            pltpu.VMEM_SHARED((N,), jnp.float32),
        ],
    )
    return kern(dst2, attr2)


_R = 1000


def _mm1_body(x_ref, w_ref, dv_ref, h_ref, hs_ref):
    h = jnp.dot(x_ref[...], w_ref[...], preferred_element_type=jnp.float32)
    h_ref[...] = h
    hs_ref[...] = h * dv_ref[...]


def _mm1(x, W, dinvb):
    return pl.pallas_call(
        _mm1_body,
        grid=(N // _R,),
        in_specs=[pl.BlockSpec((_R, D), lambda i: (i, 0)),
                  pl.BlockSpec((D, D), lambda i: (0, 0)),
                  pl.BlockSpec((_R, D), lambda i: (i, 0))],
        out_specs=[pl.BlockSpec((_R, D), lambda i: (i, 0)),
                   pl.BlockSpec((_R, D), lambda i: (i, 0))],
        out_shape=[jax.ShapeDtypeStruct((N, D), jnp.float32),
                   jax.ShapeDtypeStruct((N, D), jnp.float32)],
    )(x, W, dinvb)


def _mid_body(s_ref, h_ref, dv_ref, b_ref, w_ref, ho_ref, hso_ref):
    dv = dv_ref[...]
    xin = jnp.maximum(
        dv * (s_ref[0] + s_ref[1]) + 2.0 * dv * dv * h_ref[...] + b_ref[...],
        0.0)
    h = jnp.dot(xin, w_ref[...], preferred_element_type=jnp.float32)
    ho_ref[...] = h
    hso_ref[...] = h * dv


def _mid(s_p, h_prev, dinvb, b, W):
    return pl.pallas_call(
        _mid_body,
        grid=(N // _R,),
        in_specs=[pl.BlockSpec((NC, _R, D), lambda i: (0, i, 0)),
                  pl.BlockSpec((_R, D), lambda i: (i, 0)),
                  pl.BlockSpec((_R, D), lambda i: (i, 0)),
                  pl.BlockSpec((1, D), lambda i: (0, 0)),
                  pl.BlockSpec((D, D), lambda i: (0, 0))],
        out_specs=[pl.BlockSpec((_R, D), lambda i: (i, 0)),
                   pl.BlockSpec((_R, D), lambda i: (i, 0))],
        out_shape=[jax.ShapeDtypeStruct((N, D), jnp.float32),
                   jax.ShapeDtypeStruct((N, D), jnp.float32)],
    )(s_p, h_prev, dinvb, b, W)


def _final_body(s_ref, h_ref, dv_ref, b_ref, o_ref):
    dv = dv_ref[...]
    o_ref[...] = (dv * (s_ref[0] + s_ref[1])
                  + 2.0 * dv * dv * h_ref[...] + b_ref[...])


def _final(s_p, h_prev, dinvb, b):
    return pl.pallas_call(
        _final_body,
        grid=(N // _R,),
        in_specs=[pl.BlockSpec((NC, _R, D), lambda i: (0, i, 0)),
                  pl.BlockSpec((_R, D), lambda i: (i, 0)),
                  pl.BlockSpec((_R, D), lambda i: (i, 0)),
                  pl.BlockSpec((1, D), lambda i: (0, 0))],
        out_specs=pl.BlockSpec((_R, D), lambda i: (i, 0)),
        out_shape=jax.ShapeDtypeStruct((N, D), jnp.float32),
    )(s_p, h_prev, dinvb, b)


def kernel(x, edge_index, edge_attr, W1, b1, W2, b2, W3, b3):
    src = edge_index[0].astype(jnp.int32)
    dst = edge_index[1].astype(jnp.int32)
    E = src.shape[0]
    ep = NW * G * 8
    EP = ((E + ep - 1) // ep) * ep
    padn = EP - E
    if padn:
        pad_idx = jnp.arange(padn, dtype=jnp.int32) % N
        src = jnp.concatenate([src, pad_idx])
        dst = jnp.concatenate([dst, pad_idx])
        ea = jnp.concatenate([edge_attr.astype(jnp.float32),
                              jnp.zeros((padn,), jnp.float32)])
    else:
        ea = edge_attr.astype(jnp.float32)
    src2 = src.reshape(-1, G)
    dst2 = dst.reshape(-1, G)
    attr2 = ea.reshape(-1, G)

    degp = _degree(dst.reshape(-1, 128), ea.reshape(-1, 128))
    deg = degp[0] + degp[1] + 2.0
    dinv = jnp.where(deg > 0, jax.lax.rsqrt(deg), 0.0)
    dinvb = jnp.broadcast_to(dinv[:, None], (N, D))

    b1r = b1.reshape(1, D)
    b2r = b2.reshape(1, D)
    b3r = b3.reshape(1, D)

    h1, hs1 = _mm1(x, W1, dinvb)
    s1 = _aggregate(hs1, src2, dst2, attr2)
    h2, hs2 = _mid(s1, h1, dinvb, b1r, W2)
    s2 = _aggregate(hs2, src2, dst2, attr2)
    h3, hs3 = _mid(s2, h2, dinvb, b2r, W3)
    s3 = _aggregate(hs3, src2, dst2, attr2)
    return _final(s3, h3, dinvb, b3r)


# speedup vs baseline: 3.9454x; 1.0073x over previous
import functools

import jax
import jax.numpy as jnp
from jax import lax
from jax.experimental import pallas as pl
from jax.experimental.pallas import tpu as pltpu
from jax.experimental.pallas import tpu_sc as plsc

N = 10000
D = 128
NC = 2
NS = 16
NW = NC * NS
G = 64
NP = 10240
SP = NP // NS


def _zero_rows(rows_v):
    def zrow(i, carry):
        for q in range(8):
            rows_v[i, pl.ds(q * 16, 16)] = jnp.zeros((16,), jnp.float32)
        return carry
    lax.fori_loop(0, G, zrow, 0)


SB = 16


def _agg_body(hs_hbm, src_hbm, dst_hbm, attr_hbm, out_hbm,
              src_v, dst_v, attr_v, rows_v, acc_sh, gsem, ssem, isem):
    c = lax.axis_index("c")
    s = lax.axis_index("s")
    wid = c * NS + s
    K = src_hbm.shape[0] // NW
    NSB = K // SB

    _zero_rows(rows_v.at[0])
    base_r = s * SP
    for t in range(SP // G):
        pltpu.async_copy(rows_v.at[0], acc_sh.at[pl.ds(base_r + t * G, G)],
                         isem)
    for t in range(SP // G):
        pltpu.make_async_copy(rows_v.at[0],
                              acc_sh.at[pl.ds(base_r + t * G, G)],
                              isem).wait()
    plsc.subcore_barrier()

    idx_bufs = (src_v, dst_v, attr_v)

    def _stage(sbi, slot):
        gb = wid * K + sbi * SB
        for hb, vb in zip((src_hbm, dst_hbm, attr_hbm), idx_bufs):
            pltpu.async_copy(hb.at[pl.ds(gb, SB)], vb.at[slot], isem)

    def _stage_wait(sbi, slot):
        gb = wid * K + sbi * SB
        for hb, vb in zip((src_hbm, dst_hbm, attr_hbm), idx_bufs):
            pltpu.make_async_copy(hb.at[pl.ds(gb, SB)], vb.at[slot],
                                  isem).wait()

    def _gather(slot, gl, b):
        pltpu.async_copy(hs_hbm.at[src_v.at[slot, gl]], rows_v.at[b], gsem)

    def _gather_wait(slot, gl, b):
        pltpu.make_async_copy(hs_hbm.at[src_v.at[slot, gl]], rows_v.at[b],
                              gsem).wait()

    def _scatter(slot, gl, b):
        pltpu.async_copy(rows_v.at[b], acc_sh.at[dst_v.at[slot, gl]], ssem,
                         add=True)

    def _scatter_wait(slot, gl, b):
        pltpu.make_async_copy(rows_v.at[b], acc_sh.at[dst_v.at[slot, gl]],
                              ssem).wait()

    _stage(0, 0)
    _stage_wait(0, 0)
    _gather(0, 0, 0)
    _gather(0, 1, 1)

    def sb_pair(p, carry):
        for par in range(2):
            cur = par
            nxt = 1 - par
            sbi = p * 2 + par

            @pl.when(sbi + 1 < NSB)
            def _prefetch(_c=cur, _n=nxt):
                _stage(sbi + 1, _n)

            def inner(it, c2, _cur=cur, _nxt=nxt, _sbi=sbi):
                for u in range(4):
                    gl = it * 4 + u
                    g = _sbi * SB + gl
                    bd = (u + 2) % 4

                    if u < 2:
                        @pl.when(jnp.logical_and(g >= 2, gl >= 2))
                        def _drain_same(_b=bd, _cur=_cur, _gl=gl):
                            _scatter_wait(_cur, _gl - 2, _b)

                        @pl.when(jnp.logical_and(g >= 2, gl < 2))
                        def _drain_prev(_b=bd, _nxt=_nxt, _gl=gl):
                            _scatter_wait(_nxt, _gl + SB - 2, _b)
                    else:
                        @pl.when(g >= 2)
                        def _drain(_b=bd, _cur=_cur, _gl=gl):
                            _scatter_wait(_cur, _gl - 2, _b)

                    if u < 2:
                        @pl.when(g + 2 < K)
                        def _issue_same(_cur=_cur, _gl=gl, _b=bd):
                            _gather(_cur, _gl + 2, _b)
                    else:
                        @pl.when(it < SB // 4 - 1)
                        def _issue_same2(_cur=_cur, _gl=gl, _b=bd):
                            _gather(_cur, _gl + 2, _b)

                        @pl.when(jnp.logical_and(it == SB // 4 - 1,
                                                 _sbi + 1 < NSB))
                        def _issue_next(_nxt=_nxt, _b=bd, _u=u, _sbi=_sbi):
                            if _u == 2:
                                _stage_wait(_sbi + 1, _nxt)
                            _gather(_nxt, _u - 2, _b)

                    _gather_wait(_cur, gl, u)

                    def scale16(q16, c3, _b=u, _gl=gl, _cur=_cur):
                        wv = attr_v[_cur, _gl, pl.ds(q16 * 16, 16)]
                        for l in range(16):
                            w = wv[l]
                            e = q16 * 16 + l
                            for q in range(8):
                                sl = pl.ds(q * 16, 16)
                                rows_v[_b, e, sl] = rows_v[_b, e, sl] * w
                        return c3
                    lax.fori_loop(0, G // 16, scale16, 0, unroll=2)

                    _scatter(_cur, gl, u)
                return c2
            lax.fori_loop(0, SB // 4, inner, 0)
        return carry
    lax.fori_loop(0, NSB // 2, sb_pair, 0)

    last_slot = (NSB - 1) % 2
    for gl in (SB - 2, SB - 1):
        _scatter_wait(last_slot, gl, gl % 4)

    plsc.subcore_barrier()
    n_out = N - (NS - 1) * SP

    @pl.when(s < NS - 1)
    def _full_out():
        pltpu.sync_copy(acc_sh.at[pl.ds(base_r, SP)],
                        out_hbm.at[c, pl.ds(base_r, SP)])

    @pl.when(s == NS - 1)
    def _tail_out():
        tail = (NS - 1) * SP
        pltpu.sync_copy(acc_sh.at[pl.ds(tail, n_out)],
                        out_hbm.at[c, pl.ds(tail, n_out)])


def _aggregate(hs, src2, dst2, attr2):
    K = src2.shape[0] // NW
    kern = pl.kernel(
        _agg_body,
        out_type=jax.ShapeDtypeStruct((NC, N, D), jnp.float32),
        mesh=plsc.VectorSubcoreMesh(core_axis_name="c", subcore_axis_name="s"),
        scratch_types=[
            pltpu.VMEM((2, SB, G), jnp.int32),
            pltpu.VMEM((2, SB, G), jnp.int32),
            pltpu.VMEM((2, SB, G), jnp.float32),
            pltpu.VMEM((4, G, D), jnp.float32),
            pltpu.VMEM_SHARED((NP, D), jnp.float32),
            pltpu.SemaphoreType.DMA,
            pltpu.SemaphoreType.DMA,
            pltpu.SemaphoreType.DMA,
        ],
    )
    return kern(hs, src2, dst2, attr2)


def _deg_body(dst_hbm, attr_hbm, out_hbm, dst_v, attr_v, zb_v, acc_sh):
    c = lax.axis_index("c")
    s = lax.axis_index("s")
    wid = c * NS + s
    K = dst_hbm.shape[0] // NW

    @pl.when(s == 0)
    def _init():
        def z(i, carry):
            zb_v[pl.ds(i * 16, 16)] = jnp.zeros((16,), jnp.float32)
            return carry
        lax.fori_loop(0, N // 16, z, 0)
        pltpu.sync_copy(zb_v, acc_sh)
    plsc.subcore_barrier()

    pltpu.sync_copy(dst_hbm.at[pl.ds(wid * K, K)], dst_v)
    pltpu.sync_copy(attr_hbm.at[pl.ds(wid * K, K)], attr_v)

    def group(j, carry):
        pltpu.sync_copy(attr_v.at[j], acc_sh.at[dst_v.at[j]], add=True)
        return carry
    lax.fori_loop(0, K, group, 0)

    plsc.subcore_barrier()

    @pl.when(s == 0)
    def _out():
        pltpu.sync_copy(acc_sh, out_hbm.at[c])


def _degree(dst2, attr2):
    K = dst2.shape[0] // NW
    GD = dst2.shape[1]
    kern = pl.kernel(
        _deg_body,
        out_type=jax.ShapeDtypeStruct((NC, N), jnp.float32),
        mesh=plsc.VectorSubcoreMesh(core_axis_name="c", subcore_axis_name="s"),
        scratch_types=[
            pltpu.VMEM((K, GD), jnp.int32),
            pltpu.VMEM((K, GD), jnp.float32),
            pltpu.VMEM((N,), jnp.float32),
            pltpu.VMEM_SHARED((N,), jnp.float32),
        ],
    )
    return kern(dst2, attr2)


_R = 1000


def _mm1_body(x_ref, w_ref, dv_ref, hs_ref):
    h = jnp.dot(x_ref[...], w_ref[...], preferred_element_type=jnp.float32)
    hs_ref[...] = h * dv_ref[...]


def _mm1(x, W, dinvb):
    return pl.pallas_call(
        _mm1_body,
        grid=(N // _R,),
        in_specs=[pl.BlockSpec((_R, D), lambda i: (i, 0)),
                  pl.BlockSpec((D, D), lambda i: (0, 0)),
                  pl.BlockSpec((_R, D), lambda i: (i, 0))],
        out_specs=pl.BlockSpec((_R, D), lambda i: (i, 0)),
        out_shape=jax.ShapeDtypeStruct((N, D), jnp.float32),
    )(x, W, dinvb)


def _mid_body(s_ref, hs_ref, dv_ref, b_ref, w_ref, hso_ref):
    dv = dv_ref[...]
    xin = jnp.maximum(
        dv * (s_ref[0] + s_ref[1]) + 2.0 * dv * hs_ref[...] + b_ref[...],
        0.0)
    h = jnp.dot(xin, w_ref[...], preferred_element_type=jnp.float32)
    hso_ref[...] = h * dv


def _mid(s_p, hs_prev, dinvb, b, W):
    return pl.pallas_call(
        _mid_body,
        grid=(N // _R,),
        in_specs=[pl.BlockSpec((NC, _R, D), lambda i: (0, i, 0)),
                  pl.BlockSpec((_R, D), lambda i: (i, 0)),
                  pl.BlockSpec((_R, D), lambda i: (i, 0)),
                  pl.BlockSpec((1, D), lambda i: (0, 0)),
                  pl.BlockSpec((D, D), lambda i: (0, 0))],
        out_specs=pl.BlockSpec((_R, D), lambda i: (i, 0)),
        out_shape=jax.ShapeDtypeStruct((N, D), jnp.float32),
    )(s_p, hs_prev, dinvb, b, W)


def _final_body(s_ref, hs_ref, dv_ref, b_ref, o_ref):
    dv = dv_ref[...]
    o_ref[...] = (dv * (s_ref[0] + s_ref[1])
                  + 2.0 * dv * hs_ref[...] + b_ref[...])


def _final(s_p, hs_prev, dinvb, b):
    return pl.pallas_call(
        _final_body,
        grid=(N // _R,),
        in_specs=[pl.BlockSpec((NC, _R, D), lambda i: (0, i, 0)),
                  pl.BlockSpec((_R, D), lambda i: (i, 0)),
                  pl.BlockSpec((_R, D), lambda i: (i, 0)),
                  pl.BlockSpec((1, D), lambda i: (0, 0))],
        out_specs=pl.BlockSpec((_R, D), lambda i: (i, 0)),
        out_shape=jax.ShapeDtypeStruct((N, D), jnp.float32),
    )(s_p, hs_prev, dinvb, b)


def kernel(x, edge_index, edge_attr, W1, b1, W2, b2, W3, b3):
    src = edge_index[0].astype(jnp.int32)
    dst = edge_index[1].astype(jnp.int32)
    E = src.shape[0]
    ep = NW * G * 8
    EP = ((E + ep - 1) // ep) * ep
    padn = EP - E
    if padn:
        pad_idx = jnp.arange(padn, dtype=jnp.int32) % N
        src = jnp.concatenate([src, pad_idx])
        dst = jnp.concatenate([dst, pad_idx])
        ea = jnp.concatenate([edge_attr.astype(jnp.float32),
                              jnp.zeros((padn,), jnp.float32)])
    else:
        ea = edge_attr.astype(jnp.float32)
    src2 = src.reshape(-1, G)
    dst2 = dst.reshape(-1, G)
    attr2 = ea.reshape(-1, G)

    degp = _degree(dst.reshape(-1, 128), ea.reshape(-1, 128))
    deg = degp[0] + degp[1] + 2.0
    dinv = jnp.where(deg > 0, jax.lax.rsqrt(deg), 0.0)
    dinvb = jnp.broadcast_to(dinv[:, None], (N, D))

    b1r = b1.reshape(1, D)
    b2r = b2.reshape(1, D)
    b3r = b3.reshape(1, D)

    hs1 = _mm1(x, W1, dinvb)
    s1 = _aggregate(hs1, src2, dst2, attr2)
    hs2 = _mid(s1, hs1, dinvb, b1r, W2)
    s2 = _aggregate(hs2, src2, dst2, attr2)
    hs3 = _mid(s2, hs2, dinvb, b2r, W3)
    s3 = _aggregate(hs3, src2, dst2, attr2)
    return _final(s3, hs3, dinvb, b3r)


# speedup vs baseline: 4.0280x; 1.0209x over previous
import functools

import jax
import jax.numpy as jnp
from jax import lax
from jax.experimental import pallas as pl
from jax.experimental.pallas import tpu as pltpu
from jax.experimental.pallas import tpu_sc as plsc

N = 10000
D = 128
NC = 2
NS = 16
NW = NC * NS
G = 64
NP = 10240
SP = NP // NS


def _zero_rows(rows_v):
    def zrow(i, carry):
        for q in range(8):
            rows_v[i, pl.ds(q * 16, 16)] = jnp.zeros((16,), jnp.float32)
        return carry
    lax.fori_loop(0, G, zrow, 0)


SB = 16


def _agg_body(hs_hbm, src_hbm, dst_hbm, attr_hbm, out_hbm,
              src_v, dst_v, attr_v, rows_v, acc_sh, gsem, ssem, isem):
    c = lax.axis_index("c")
    s = lax.axis_index("s")
    wid = c * NS + s
    K = src_hbm.shape[0] // NW
    NSB = K // SB

    _zero_rows(rows_v.at[0])
    base_r = s * SP
    for t in range(SP // G):
        pltpu.async_copy(rows_v.at[0], acc_sh.at[pl.ds(base_r + t * G, G)],
                         isem)
    for t in range(SP // G):
        pltpu.make_async_copy(rows_v.at[0],
                              acc_sh.at[pl.ds(base_r + t * G, G)],
                              isem).wait()
    plsc.subcore_barrier()

    idx_bufs = (src_v, dst_v, attr_v)

    def _stage(sbi, slot):
        gb = wid * K + sbi * SB
        for hb, vb in zip((src_hbm, dst_hbm, attr_hbm), idx_bufs):
            pltpu.async_copy(hb.at[pl.ds(gb, SB)], vb.at[slot], isem)

    def _stage_wait(sbi, slot):
        gb = wid * K + sbi * SB
        for hb, vb in zip((src_hbm, dst_hbm, attr_hbm), idx_bufs):
            pltpu.make_async_copy(hb.at[pl.ds(gb, SB)], vb.at[slot],
                                  isem).wait()

    def _gather(slot, gl, b):
        pltpu.async_copy(hs_hbm.at[src_v.at[slot, gl]], rows_v.at[b], gsem)

    def _gather_wait(slot, gl, b):
        pltpu.make_async_copy(hs_hbm.at[src_v.at[slot, gl]], rows_v.at[b],
                              gsem).wait()

    def _scatter(slot, gl, b):
        pltpu.async_copy(rows_v.at[b], acc_sh.at[dst_v.at[slot, gl]], ssem,
                         add=True)

    def _scatter_wait(slot, gl, b):
        pltpu.make_async_copy(rows_v.at[b], acc_sh.at[dst_v.at[slot, gl]],
                              ssem).wait()

    _stage(0, 0)
    _stage_wait(0, 0)
    _gather(0, 0, 0)
    _gather(0, 1, 1)

    def sb_pair(p, carry):
        for par in range(2):
            cur = par
            nxt = 1 - par
            sbi = p * 2 + par

            @pl.when(sbi + 1 < NSB)
            def _prefetch(_c=cur, _n=nxt):
                _stage(sbi + 1, _n)

            def inner(it, c2, _cur=cur, _nxt=nxt, _sbi=sbi):
                for u in range(4):
                    gl = it * 4 + u
                    g = _sbi * SB + gl
                    bd = (u + 2) % 4

                    if u < 2:
                        @pl.when(jnp.logical_and(g >= 2, gl >= 2))
                        def _drain_same(_b=bd, _cur=_cur, _gl=gl):
                            _scatter_wait(_cur, _gl - 2, _b)

                        @pl.when(jnp.logical_and(g >= 2, gl < 2))
                        def _drain_prev(_b=bd, _nxt=_nxt, _gl=gl):
                            _scatter_wait(_nxt, _gl + SB - 2, _b)
                    else:
                        @pl.when(g >= 2)
                        def _drain(_b=bd, _cur=_cur, _gl=gl):
                            _scatter_wait(_cur, _gl - 2, _b)

                    if u < 2:
                        @pl.when(g + 2 < K)
                        def _issue_same(_cur=_cur, _gl=gl, _b=bd):
                            _gather(_cur, _gl + 2, _b)
                    else:
                        @pl.when(it < SB // 4 - 1)
                        def _issue_same2(_cur=_cur, _gl=gl, _b=bd):
                            _gather(_cur, _gl + 2, _b)

                        @pl.when(jnp.logical_and(it == SB // 4 - 1,
                                                 _sbi + 1 < NSB))
                        def _issue_next(_nxt=_nxt, _b=bd, _u=u, _sbi=_sbi):
                            if _u == 2:
                                _stage_wait(_sbi + 1, _nxt)
                            _gather(_nxt, _u - 2, _b)

                    _gather_wait(_cur, gl, u)

                    def scale16(q16, c3, _b=u, _gl=gl, _cur=_cur):
                        wv = attr_v[_cur, _gl, pl.ds(q16 * 16, 16)]
                        for l in range(16):
                            w = wv[l]
                            e = q16 * 16 + l
                            for q in range(8):
                                sl = pl.ds(q * 16, 16)
                                rows_v[_b, e, sl] = rows_v[_b, e, sl] * w
                        return c3
                    lax.fori_loop(0, G // 16, scale16, 0, unroll=2)

                    _scatter(_cur, gl, u)
                return c2
            lax.fori_loop(0, SB // 4, inner, 0)
        return carry
    lax.fori_loop(0, NSB // 2, sb_pair, 0)

    last_slot = (NSB - 1) % 2
    for gl in (SB - 2, SB - 1):
        _scatter_wait(last_slot, gl, gl % 4)

    plsc.subcore_barrier()
    n_out = N - (NS - 1) * SP

    @pl.when(s < NS - 1)
    def _full_out():
        pltpu.sync_copy(acc_sh.at[pl.ds(base_r, SP)],
                        out_hbm.at[c, pl.ds(base_r, SP)])

    @pl.when(s == NS - 1)
    def _tail_out():
        tail = (NS - 1) * SP
        pltpu.sync_copy(acc_sh.at[pl.ds(tail, n_out)],
                        out_hbm.at[c, pl.ds(tail, n_out)])


def _aggregate(hs, src2, dst2, attr2):
    K = src2.shape[0] // NW
    kern = pl.kernel(
        _agg_body,
        out_type=jax.ShapeDtypeStruct((NC, N, D), jnp.float32),
        mesh=plsc.VectorSubcoreMesh(core_axis_name="c", subcore_axis_name="s"),
        scratch_types=[
            pltpu.VMEM((2, SB, G), jnp.int32),
            pltpu.VMEM((2, SB, G), jnp.int32),
            pltpu.VMEM((2, SB, G), jnp.float32),
            pltpu.VMEM((4, G, D), jnp.float32),
            pltpu.VMEM_SHARED((NP, D), jnp.float32),
            pltpu.SemaphoreType.DMA,
            pltpu.SemaphoreType.DMA,
            pltpu.SemaphoreType.DMA,
        ],
    )
    return kern(hs, src2, dst2, attr2)


def _deg_body(dst_hbm, attr_hbm, out_hbm, dst_v, attr_v, zb_v, acc_sh):
    c = lax.axis_index("c")
    s = lax.axis_index("s")
    wid = c * NS + s
    K = dst_hbm.shape[0] // NW

    @pl.when(s == 0)
    def _init():
        def z(i, carry):
            zb_v[pl.ds(i * 16, 16)] = jnp.zeros((16,), jnp.float32)
            return carry
        lax.fori_loop(0, N // 16, z, 0)
        pltpu.sync_copy(zb_v, acc_sh)
    plsc.subcore_barrier()

    pltpu.sync_copy(dst_hbm.at[pl.ds(wid * K, K)], dst_v)
    pltpu.sync_copy(attr_hbm.at[pl.ds(wid * K, K)], attr_v)

    def group(j, carry):
        pltpu.sync_copy(attr_v.at[j], acc_sh.at[dst_v.at[j]], add=True)
        return carry
    lax.fori_loop(0, K, group, 0)

    plsc.subcore_barrier()

    @pl.when(s == 0)
    def _out():
        pltpu.sync_copy(acc_sh, out_hbm.at[c])


def _degree(dst2, attr2):
    K = dst2.shape[0] // NW
    GD = dst2.shape[1]
    kern = pl.kernel(
        _deg_body,
        out_type=jax.ShapeDtypeStruct((NC, N), jnp.float32),
        mesh=plsc.VectorSubcoreMesh(core_axis_name="c", subcore_axis_name="s"),
        scratch_types=[
            pltpu.VMEM((K, GD), jnp.int32),
            pltpu.VMEM((K, GD), jnp.float32),
            pltpu.VMEM((N,), jnp.float32),
            pltpu.VMEM_SHARED((N,), jnp.float32),
        ],
    )
    return kern(dst2, attr2)


_R = 2000


def _mm1_body(x_ref, w_ref, dv_ref, hs_ref):
    h = jnp.dot(x_ref[...], w_ref[...], preferred_element_type=jnp.float32)
    hs_ref[...] = h * dv_ref[...]


def _mm1(x, W, dinvb):
    return pl.pallas_call(
        _mm1_body,
        grid=(N // _R,),
        in_specs=[pl.BlockSpec((_R, D), lambda i: (i, 0)),
                  pl.BlockSpec((D, D), lambda i: (0, 0)),
                  pl.BlockSpec((_R, D), lambda i: (i, 0))],
        out_specs=pl.BlockSpec((_R, D), lambda i: (i, 0)),
        out_shape=jax.ShapeDtypeStruct((N, D), jnp.float32),
    )(x, W, dinvb)


def _mid_body(s_ref, hs_ref, dv_ref, b_ref, w_ref, hso_ref):
    dv = dv_ref[...]
    xin = jnp.maximum(
        dv * (s_ref[0] + s_ref[1]) + 2.0 * dv * hs_ref[...] + b_ref[...],
        0.0)
    h = jnp.dot(xin, w_ref[...], preferred_element_type=jnp.float32)
    hso_ref[...] = h * dv


def _mid(s_p, hs_prev, dinvb, b, W):
    return pl.pallas_call(
        _mid_body,
        grid=(N // _R,),
        in_specs=[pl.BlockSpec((NC, _R, D), lambda i: (0, i, 0)),
                  pl.BlockSpec((_R, D), lambda i: (i, 0)),
                  pl.BlockSpec((_R, D), lambda i: (i, 0)),
                  pl.BlockSpec((1, D), lambda i: (0, 0)),
                  pl.BlockSpec((D, D), lambda i: (0, 0))],
        out_specs=pl.BlockSpec((_R, D), lambda i: (i, 0)),
        out_shape=jax.ShapeDtypeStruct((N, D), jnp.float32),
    )(s_p, hs_prev, dinvb, b, W)


def _final_body(s_ref, hs_ref, dv_ref, b_ref, o_ref):
    dv = dv_ref[...]
    o_ref[...] = (dv * (s_ref[0] + s_ref[1])
                  + 2.0 * dv * hs_ref[...] + b_ref[...])


def _final(s_p, hs_prev, dinvb, b):
    return pl.pallas_call(
        _final_body,
        grid=(N // _R,),
        in_specs=[pl.BlockSpec((NC, _R, D), lambda i: (0, i, 0)),
                  pl.BlockSpec((_R, D), lambda i: (i, 0)),
                  pl.BlockSpec((_R, D), lambda i: (i, 0)),
                  pl.BlockSpec((1, D), lambda i: (0, 0))],
        out_specs=pl.BlockSpec((_R, D), lambda i: (i, 0)),
        out_shape=jax.ShapeDtypeStruct((N, D), jnp.float32),
    )(s_p, hs_prev, dinvb, b)


def kernel(x, edge_index, edge_attr, W1, b1, W2, b2, W3, b3):
    src = edge_index[0].astype(jnp.int32)
    dst = edge_index[1].astype(jnp.int32)
    E = src.shape[0]
    ep = NW * G * 8
    EP = ((E + ep - 1) // ep) * ep
    padn = EP - E
    if padn:
        pad_idx = jnp.arange(padn, dtype=jnp.int32) % N
        src = jnp.concatenate([src, pad_idx])
        dst = jnp.concatenate([dst, pad_idx])
        ea = jnp.concatenate([edge_attr.astype(jnp.float32),
                              jnp.zeros((padn,), jnp.float32)])
    else:
        ea = edge_attr.astype(jnp.float32)
    src2 = src.reshape(-1, G)
    dst2 = dst.reshape(-1, G)
    attr2 = ea.reshape(-1, G)

    degp = _degree(dst.reshape(-1, 128), ea.reshape(-1, 128))
    deg = degp[0] + degp[1] + 2.0
    dinv = jnp.where(deg > 0, jax.lax.rsqrt(deg), 0.0)
    dinvb = jnp.broadcast_to(dinv[:, None], (N, D))

    b1r = b1.reshape(1, D)
    b2r = b2.reshape(1, D)
    b3r = b3.reshape(1, D)

    hs1 = _mm1(x, W1, dinvb)
    s1 = _aggregate(hs1, src2, dst2, attr2)
    hs2 = _mid(s1, hs1, dinvb, b1r, W2)
    s2 = _aggregate(hs2, src2, dst2, attr2)
    hs3 = _mid(s2, hs2, dinvb, b2r, W3)
    s3 = _aggregate(hs3, src2, dst2, attr2)
    return _final(s3, hs3, dinvb, b3r)
